# baseline (speedup 1.0000x reference)
import jax
import jax.numpy as jnp
from jax.experimental import pallas as pl
from jax.experimental.pallas import tpu as pltpu
from jax.sharding import PartitionSpec as P

ALPHA_ = 0.1
BM = 512

_N_SHARDS = max(d for d in (1, 2, 4) if len(jax.devices()) >= d and 4096 % (d * BM) == 0)
_MESH = jax.make_mesh((_N_SHARDS,), ("i",))


def _prop_kernel(adj_ref, x_ref, h_ref, o_ref):
    o_ref[...] = (1.0 - ALPHA_) * jnp.dot(
        adj_ref[...], x_ref[...], preferred_element_type=jnp.float32
    ) + ALPHA_ * h_ref[...]


def _prop_shard(adj, x, h):
    m, n = adj.shape
    d = x.shape[1]
    nm = m // BM
    return pl.pallas_call(
        _prop_kernel,
        grid=(nm,),
        in_specs=[
            pl.BlockSpec((BM, n), lambda i: (i, 0)),
            pl.BlockSpec((n, d), lambda i: (0, 0)),
            pl.BlockSpec((BM, d), lambda i: (i, 0)),
        ],
        out_specs=pl.BlockSpec((BM, d), lambda i: (i, 0)),
        out_shape=jax.ShapeDtypeStruct((m, d), jnp.float32),
        compiler_params=pltpu.CompilerParams(
            dimension_semantics=("parallel",),
        ),
    )(adj, x, h)


@jax.jit
def kernel(x, adj, h):
    adj = jax.reshard(adj, jax.sharding.NamedSharding(_MESH, P("i", None)))
    x = jax.reshard(x, jax.sharding.NamedSharding(_MESH, P(None, None)))
    h = jax.reshard(h, jax.sharding.NamedSharding(_MESH, P("i", None)))
    return jax.shard_map(
        _prop_shard,
        mesh=_MESH,
        in_specs=(P("i", None), P(None, None), P("i", None)),
        out_specs=P("i", None),
        check_vma=False,
    )(adj, x, h)


# speedup vs baseline: 14.6573x; 14.6573x over previous
import jax
import jax.numpy as jnp
from jax.experimental import pallas as pl
from jax.experimental.pallas import tpu as pltpu

ALPHA_ = 0.1
BM = 512
NSPLIT = 2


def _prop_kernel(adj_ref, x_ref, h_ref, o_ref):
    j = pl.program_id(1)
    kw = adj_ref.shape[1]
    part = jnp.dot(
        adj_ref[...],
        x_ref[pl.ds(j * kw, kw), :],
        preferred_element_type=jnp.float32,
    )

    @pl.when(j == 0)
    def _first():
        o_ref[...] = part

    @pl.when(jnp.logical_and(j > 0, j < NSPLIT - 1))
    def _middle():
        o_ref[...] += part

    @pl.when(j == NSPLIT - 1)
    def _last():
        o_ref[...] = (1.0 - ALPHA_) * (o_ref[...] + part) + ALPHA_ * h_ref[...]


@jax.jit
def kernel(x, adj, h):
    n, d = x.shape
    nm = n // BM
    bk = n // NSPLIT
    return pl.pallas_call(
        _prop_kernel,
        grid=(nm, NSPLIT),
        in_specs=[
            pl.BlockSpec((BM, bk), lambda i, j: (i, j)),
            pl.BlockSpec((n, d), lambda i, j: (0, 0)),
            pl.BlockSpec((BM, d), lambda i, j: (jnp.where(j == NSPLIT - 1, i, 0), 0)),
        ],
        out_specs=pl.BlockSpec((BM, d), lambda i, j: (i, 0)),
        out_shape=jax.ShapeDtypeStruct((n, d), jnp.float32),
        compiler_params=pltpu.CompilerParams(
            dimension_semantics=("parallel", "arbitrary"),
        ),
    )(adj, x, h)


# speedup vs baseline: 17.9904x; 1.2274x over previous
import jax
import jax.numpy as jnp
from jax.experimental import pallas as pl
from jax.experimental.pallas import tpu as pltpu

ALPHA_ = 0.1
BM = 512


def _prop_kernel(adj_ref, x_ref, h_ref, o_ref):
    i = pl.program_id(0)
    o_ref[...] = (1.0 - ALPHA_) * jnp.dot(
        adj_ref[...], x_ref[...], preferred_element_type=jnp.float32
    ) + ALPHA_ * h_ref[pl.ds(i * BM, BM), :]


@jax.jit
def kernel(x, adj, h):
    n, d = x.shape
    nm = n // BM
    return pl.pallas_call(
        _prop_kernel,
        grid=(nm,),
        in_specs=[
            pl.BlockSpec((BM, n), lambda i: (i, 0)),
            pl.BlockSpec((n, d), lambda i: (0, 0)),
            pl.BlockSpec((n, d), lambda i: (0, 0)),
        ],
        out_specs=pl.BlockSpec((BM, d), lambda i: (i, 0)),
        out_shape=jax.ShapeDtypeStruct((n, d), jnp.float32),
        compiler_params=pltpu.CompilerParams(
            dimension_semantics=("parallel",),
        ),
    )(adj, x, h)
